# baseline (speedup 1.0000x reference)
import functools

import jax
import jax.numpy as jnp
from jax import lax
from jax.experimental import pallas as pl
from jax.experimental.pallas import tpu as pltpu
from jax.experimental.pallas import tpu_sc as plsc

_B = 16384
_F = 26
_VOCAB = 100000
_D = 16
_ED = _F * _D
_EP = 512
_K = 13
_EPS = 1e-5
_ROWS = _B * _F

_NC, _NS = 2, 16
_NW = _NC * _NS
_RB = _B // 8
_RBW = _RB // _NW
_RPW = _ROWS // _NW
_R16 = 8
_CH = _R16 * 208
_GPC = _CH // 128
_NCH = _RBW // _R16
_IPW = _RPW // 128
_NFC = (8, 8, 8, 2)
_OFFC = (0, 64, 128, 192)

_BS = 1024
_NB = _B // _BS
_BR = _BS // 8


def _gather_body(table_hbm, idx_hbm, out_hbm, idx_v, buf0, buf1, pk, sem0,
                 sem1):
    wid = lax.axis_index("s") * _NC + lax.axis_index("c")
    pltpu.sync_copy(idx_hbm.at[pl.ds(wid * _IPW, _IPW)], idx_v)
    rb0 = wid * _RBW
    bufs = (buf0, buf1)
    sems = (sem0, sem1)
    copies = [None] * _NCH
    zed = jnp.zeros((_D,), jnp.float32)

    def start(c):
        b = c % 2
        cps = []
        for j in range(_GPC):
            cps.append(pltpu.async_copy(
                table_hbm.at[idx_v.at[c * _GPC + j]],
                bufs[b].at[pl.ds(j * 128, 128)],
                sems[b]))
        copies[c] = cps

    def repack(buf, rb16):
        g0 = rb16 * 208
        for c in range(4):
            for s in range(8):
                for j in range(_NFC[c]):
                    v = buf[g0 + _OFFC[c] + s * _NFC[c] + j, :]
                    pk[rb16, c * 8 + s, pl.ds(j * _D, _D)] = v
                if c == 3:
                    for j in range(2, 8):
                        pk[rb16, c * 8 + s, pl.ds(j * _D, _D)] = zed

    start(0)
    for c in range(_NCH):
        if c + 1 < _NCH:
            start(c + 1)
        for cp in copies[c]:
            cp.wait()
        buf = bufs[c % 2]

        @pl.loop(0, _R16)
        def _rblk(rb16):
            repack(buf, rb16)

        pltpu.sync_copy(pk, out_hbm.at[pl.ds(rb0 + c * _R16, _R16)])


@functools.cache
def _gather_kernel_fn():
    mesh = plsc.VectorSubcoreMesh(core_axis_name="c", subcore_axis_name="s")
    return pl.kernel(
        _gather_body,
        out_type=jax.ShapeDtypeStruct((_RB, 32, 128), jnp.float32),
        mesh=mesh,
        scratch_types=[
            pltpu.VMEM((_IPW, 128), jnp.int32),
            pltpu.VMEM((_CH, _D), jnp.float32),
            pltpu.VMEM((_CH, _D), jnp.float32),
            pltpu.VMEM((_R16, 32, 128), jnp.float32),
            pltpu.SemaphoreType.DMA,
            pltpu.SemaphoreType.DMA,
        ],
        compiler_params=pltpu.CompilerParams(use_tc_tiling_on_sc=False),
    )


def _gather_kernel(table, idx2):
    return _gather_kernel_fn()(table, idx2)


def _assemble(e4):
    folds = [
        jnp.reshape(e4[:, 8 * c:8 * c + 8, :], (_BS, 128)) for c in range(4)
    ]
    return jnp.concatenate(folds, axis=1)


def _field_stats_body(e_ref, acc_ref):
    i = pl.program_id(0)
    blk = _assemble(e_ref[...])
    s1 = jnp.sum(blk, axis=0, keepdims=True)
    s2 = jnp.sum(blk * blk, axis=0, keepdims=True)
    both = jnp.concatenate([s1, s2], axis=0)

    @pl.when(i == 0)
    def _init():
        acc_ref[...] = both

    @pl.when(i > 0)
    def _acc():
        acc_ref[...] += both


def _ctrl_stats_body(e_ref, s_ref, t_ref, cw_ref, cb_ref, acc_ref):
    i = pl.program_id(0)
    en = _assemble(e_ref[...]) * s_ref[...] + t_ref[...]
    w = jnp.dot(en, cw_ref[...], preferred_element_type=jnp.float32) + cb_ref[...]
    s1 = jnp.sum(w, axis=0, keepdims=True)
    s2 = jnp.sum(w * w, axis=0, keepdims=True)
    both = jnp.concatenate([s1, s2], axis=0)

    @pl.when(i == 0)
    def _init():
        acc_ref[...] = both

    @pl.when(i > 0)
    def _acc():
        acc_ref[...] += both


def _main_body(e_ref, s_ref, t_ref, cw_ref, cb_ref, wa_ref, wb_ref, ex_ref,
               vw_ref, cst_ref, w1_ref, b1_ref, h1_ref, p1_ref, acc_ref):
    i = pl.program_id(0)
    en = _assemble(e_ref[...]) * s_ref[...] + t_ref[...]
    w = jnp.dot(en, cw_ref[...], preferred_element_type=jnp.float32) + cb_ref[...]
    wn = jnp.maximum(w * wa_ref[...] + wb_ref[...], 0.0)
    cnt = jnp.zeros_like(wn)
    for g in range(_F):
        cnt = cnt + (wn[:, g:g + 1] > wn).astype(jnp.float32)
    sel = (cnt < float(_K)) & (wn > 0.0)
    wsel = jnp.where(sel, wn, 0.0)
    mask = wsel / jnp.sum(wsel, axis=1, keepdims=True)
    x0 = en * jnp.dot(mask, ex_ref[...], preferred_element_type=jnp.float32)
    vv = jnp.dot(x0, vw_ref[...], preferred_element_type=jnp.float32)
    a1 = 1.0 + vv[:, 0:1]
    a2 = a1 * (1.0 + vv[:, 1:2]) + cst_ref[0:1, 0:1]
    a3 = a2 * (1.0 + vv[:, 2:3]) + cst_ref[0:1, 1:2]
    p1_ref[...] = a3 * vv[:, 3:4] + cst_ref[0:1, 2:3]
    h1 = jnp.dot(x0, w1_ref[...], preferred_element_type=jnp.float32) + b1_ref[...]
    h1_ref[...] = h1
    s1 = jnp.sum(h1, axis=0, keepdims=True)
    s2 = jnp.sum(h1 * h1, axis=0, keepdims=True)
    both = jnp.concatenate([s1, s2], axis=0)

    @pl.when(i == 0)
    def _init():
        acc_ref[...] = both

    @pl.when(i > 0)
    def _acc():
        acc_ref[...] += both


def _mlp2_body(h1_ref, a_ref, c_ref, w2_ref, b2_ref, h2_ref, acc_ref):
    i = pl.program_id(0)
    h = jnp.maximum(h1_ref[...] * a_ref[...] + c_ref[...], 0.0)
    h2 = jnp.dot(h, w2_ref[...], preferred_element_type=jnp.float32) + b2_ref[...]
    h2_ref[...] = h2
    s1 = jnp.sum(h2, axis=0, keepdims=True)
    s2 = jnp.sum(h2 * h2, axis=0, keepdims=True)
    both = jnp.concatenate([s1, s2], axis=0)

    @pl.when(i == 0)
    def _init():
        acc_ref[...] = both

    @pl.when(i > 0)
    def _acc():
        acc_ref[...] += both


def _final_body(h2_ref, p1_ref, a_ref, c_ref, wl_ref, out_ref):
    h = jnp.maximum(h2_ref[...] * a_ref[...] + c_ref[...], 0.0)
    p = jnp.dot(h, wl_ref[...], preferred_element_type=jnp.float32) + p1_ref[...]
    out_ref[...] = jax.nn.sigmoid(p)


def kernel(x, table, bn_gamma, bn_beta, ctrl_W, ctrl_b, ctrl_bn_g, ctrl_bn_b,
           cross_W, cross_b, mlp_W1, mlp_b1, bn1_g, bn1_b, mlp_W2, mlp_b2,
           bn2_g, bn2_b, lin_W, lin_b):
    xoff = x + (jnp.arange(_F, dtype=jnp.int32) * _VOCAB)[None, :]
    v3 = xoff.reshape(_RB, 8, _F)
    parts = [
        v3[:, :, 8 * c:8 * c + _NFC[c]].reshape(_RB, 8 * _NFC[c])
        for c in range(4)
    ]
    idx2 = jnp.concatenate(parts, axis=1).reshape(_ROWS // 128, 128)
    e4 = _gather_kernel(table, idx2)

    eblock = lambda blk: pl.BlockSpec((blk, 32, 128), lambda i: (i, 0, 0))
    const_spec = lambda shape: pl.BlockSpec(shape, lambda i: tuple(0 for _ in shape))

    stats = pl.pallas_call(
        _field_stats_body,
        grid=(_NB,),
        in_specs=[eblock(_BR)],
        out_specs=pl.BlockSpec((2, _EP), lambda i: (0, 0)),
        out_shape=jax.ShapeDtypeStruct((2, _EP), jnp.float32),
    )(e4)
    n = float(_B * _D)
    fsum = stats[0, :_ED].reshape(_F, _D).sum(axis=1)
    fsq = stats[1, :_ED].reshape(_F, _D).sum(axis=1)
    fm = fsum / n
    fv = fsq / n - fm * fm
    sf = bn_gamma / jnp.sqrt(fv + _EPS)
    tf = bn_beta - fm * sf
    pad = jnp.zeros((_EP - _ED,), jnp.float32)
    s_vec = jnp.concatenate([jnp.repeat(sf, _D), pad])[None, :]
    t_vec = jnp.concatenate([jnp.repeat(tf, _D), pad])[None, :]

    cw512 = jnp.pad(ctrl_W, ((0, _EP - _ED), (0, 0)))

    wstats = pl.pallas_call(
        _ctrl_stats_body,
        grid=(_NB,),
        in_specs=[
            eblock(_BR),
            const_spec((1, _EP)),
            const_spec((1, _EP)),
            const_spec((_EP, _F)),
            const_spec((1, _F)),
        ],
        out_specs=pl.BlockSpec((2, _F), lambda i: (0, 0)),
        out_shape=jax.ShapeDtypeStruct((2, _F), jnp.float32),
    )(e4, s_vec, t_vec, cw512, ctrl_b[None, :])
    wm = wstats[0] / _B
    wv = wstats[1] / _B - wm * wm
    wa = ctrl_bn_g[None, :] / jnp.sqrt(wv + _EPS)
    wb = ctrl_bn_b[None, :] - wm * wa

    ex = (jnp.arange(_EP, dtype=jnp.int32)[None, :] // _D
          == jnp.arange(_F, dtype=jnp.int32)[:, None]).astype(jnp.float32)
    lin_top = lin_W[:_ED, :]
    vw = jnp.pad(jnp.concatenate([cross_W.T, lin_top], axis=1),
                 ((0, _EP - _ED), (0, 0)))
    w1512 = jnp.pad(mlp_W1, ((0, _EP - _ED), (0, 0)))
    c01 = jnp.dot(cross_b[0], cross_W[1])
    c2s = jnp.dot(cross_b[0] + cross_b[1], cross_W[2])
    pconst = jnp.dot(cross_b[0] + cross_b[1] + cross_b[2], lin_top[:, 0]) + lin_b[0]
    cst = jnp.stack([c01, c2s, pconst, jnp.float32(0)])[None, :]

    h1, p1, h1stats = pl.pallas_call(
        _main_body,
        grid=(_NB,),
        in_specs=[
            eblock(_BR),
            const_spec((1, _EP)),
            const_spec((1, _EP)),
            const_spec((_EP, _F)),
            const_spec((1, _F)),
            const_spec((1, _F)),
            const_spec((1, _F)),
            const_spec((_F, _EP)),
            const_spec((_EP, 4)),
            const_spec((1, 4)),
            const_spec((_EP, 128)),
            const_spec((1, 128)),
        ],
        out_specs=[
            pl.BlockSpec((_BS, 128), lambda i: (i, 0)),
            pl.BlockSpec((_BS, 1), lambda i: (i, 0)),
            pl.BlockSpec((2, 128), lambda i: (0, 0)),
        ],
        out_shape=[
            jax.ShapeDtypeStruct((_B, 128), jnp.float32),
            jax.ShapeDtypeStruct((_B, 1), jnp.float32),
            jax.ShapeDtypeStruct((2, 128), jnp.float32),
        ],
    )(e4, s_vec, t_vec, cw512, ctrl_b[None, :], wa, wb, ex, vw, cst,
      w1512, mlp_b1[None, :])
    h1m = h1stats[0] / _B
    h1v = h1stats[1] / _B - h1m * h1m
    a1 = bn1_g[None, :] / jnp.sqrt(h1v + _EPS)
    c1 = bn1_b[None, :] - h1m * a1

    h2, h2stats = pl.pallas_call(
        _mlp2_body,
        grid=(_NB,),
        in_specs=[
            pl.BlockSpec((_BS, 128), lambda i: (i, 0)),
            const_spec((1, 128)),
            const_spec((1, 128)),
            const_spec((128, 64)),
            const_spec((1, 64)),
        ],
        out_specs=[
            pl.BlockSpec((_BS, 64), lambda i: (i, 0)),
            pl.BlockSpec((2, 64), lambda i: (0, 0)),
        ],
        out_shape=[
            jax.ShapeDtypeStruct((_B, 64), jnp.float32),
            jax.ShapeDtypeStruct((2, 64), jnp.float32),
        ],
    )(h1, a1, c1, mlp_W2, mlp_b2[None, :])
    h2m = h2stats[0] / _B
    h2v = h2stats[1] / _B - h2m * h2m
    a2 = bn2_g[None, :] / jnp.sqrt(h2v + _EPS)
    c2 = bn2_b[None, :] - h2m * a2

    out = pl.pallas_call(
        _final_body,
        grid=(_NB,),
        in_specs=[
            pl.BlockSpec((_BS, 64), lambda i: (i, 0)),
            pl.BlockSpec((_BS, 1), lambda i: (i, 0)),
            const_spec((1, 64)),
            const_spec((1, 64)),
            const_spec((64, 1)),
        ],
        out_specs=pl.BlockSpec((_BS, 1), lambda i: (i, 0)),
        out_shape=jax.ShapeDtypeStruct((_B, 1), jnp.float32),
    )(h2, p1, a2, c2, lin_W[_ED:, :])
    return out.reshape(_B)
